# baseline (speedup 1.0000x reference)
import functools

import jax
import jax.numpy as jnp
from jax import lax
from jax.experimental import pallas as pl
from jax.experimental.pallas import tpu as pltpu
from jax.experimental.pallas import tpu_sc as plsc

_LANES = 16
_CHUNK = 128


def _rsqrt_newton(x):
    xi = lax.bitcast_convert_type(x, jnp.int32)
    yi = jnp.int32(0x5F3759DF) - lax.shift_right_logical(xi, 1)
    y = lax.bitcast_convert_type(yi, jnp.float32)
    xh = x * jnp.float32(0.5)
    y = y * (jnp.float32(1.5) - xh * y * y)
    y = y * (jnp.float32(1.5) - xh * y * y)
    return y


def kernel(entity_embeddings, relation_embeddings, source, r, target_entity_type=1):
    B = source.shape[0]
    D = entity_embeddings.shape[1]
    NC, NS = 2, 16
    NW = NC * NS
    assert B % (NW * _CHUNK) == 0 and D % _LANES == 0
    b_per_w = B // NW
    n_chunks = b_per_w // _CHUNK
    n_vec = D // _LANES

    sign = jnp.where(jnp.asarray(target_entity_type) != 0,
                     jnp.float32(1.0), jnp.float32(-1.0))
    sign_arr = jnp.broadcast_to(sign, (_LANES,)).astype(jnp.float32)

    mesh = plsc.VectorSubcoreMesh(core_axis_name="c", subcore_axis_name="s")

    @functools.partial(
        pl.kernel,
        mesh=mesh,
        compiler_params=pltpu.CompilerParams(needs_layout_passes=False,
                                             use_tc_tiling_on_sc=False),
        out_type=jax.ShapeDtypeStruct((B, D), jnp.float32),
        scratch_types=[
            pltpu.VMEM((n_chunks, _CHUNK), jnp.int32),
            pltpu.VMEM((n_chunks, _CHUNK), jnp.int32),
            pltpu.VMEM((b_per_w, D), jnp.float32),
            pltpu.VMEM((b_per_w, D), jnp.float32),
            pltpu.VMEM((b_per_w, D), jnp.float32),
            pltpu.VMEM((_LANES,), jnp.float32),
            pltpu.SemaphoreType.DMA,
            pltpu.SemaphoreType.DMA,
        ],
    )
    def _translate(ent_hbm, rel_hbm, src_hbm, r_hbm, sign_hbm, out_hbm,
                   idx_s, idx_r, rows_e, rows_r, rows_o, sign_v, sem_e, sem_r):
        wid = lax.axis_index("s") * NC + lax.axis_index("c")
        base = wid * b_per_w

        pltpu.sync_copy(sign_hbm, sign_v)
        for j in range(n_chunks):
            pltpu.sync_copy(src_hbm.at[pl.ds(base + j * _CHUNK, _CHUNK)],
                            idx_s.at[j])
            pltpu.sync_copy(r_hbm.at[pl.ds(base + j * _CHUNK, _CHUNK)],
                            idx_r.at[j])

        copies = []
        for j in range(n_chunks):
            copies.append(pltpu.async_copy(
                ent_hbm.at[idx_s.at[j]],
                rows_e.at[pl.ds(j * _CHUNK, _CHUNK)], sem_e))
            copies.append(pltpu.async_copy(
                rel_hbm.at[idx_r.at[j]],
                rows_r.at[pl.ds(j * _CHUNK, _CHUNK)], sem_r))
        for c in copies:
            c.wait()

        sv = sign_v[...]

        def row_body(i, carry):
            e = [rows_e[i, pl.ds(j * _LANES, _LANES)] for j in range(n_vec)]
            rr = [rows_r[i, pl.ds(j * _LANES, _LANES)] for j in range(n_vec)]
            se = e[0] * e[0]
            sr = rr[0] * rr[0]
            for j in range(1, n_vec):
                se = se + e[j] * e[j]
                sr = sr + rr[j] * rr[j]
            te = jnp.maximum(jnp.sum(se), jnp.float32(1e-12))
            tr = jnp.maximum(jnp.sum(sr), jnp.float32(1e-12))
            inv_e = _rsqrt_newton(te)
            inv_rv = sv * _rsqrt_newton(tr)
            for j in range(n_vec):
                rows_o[i, pl.ds(j * _LANES, _LANES)] = e[j] * inv_e + rr[j] * inv_rv
            return carry

        lax.fori_loop(0, b_per_w, row_body, 0)

        pltpu.sync_copy(rows_o, out_hbm.at[pl.ds(base, b_per_w)])

    return _translate(entity_embeddings, relation_embeddings,
                      source.astype(jnp.int32), r.astype(jnp.int32), sign_arr)
